# Initial kernel scaffold; baseline (speedup 1.0000x reference)
import jax
import jax.numpy as jnp
from jax.experimental import pallas as pl


def kernel(input, edge_index, edge_types, weight_vector, bias_vector):
    raise NotImplementedError("write your pallas kernel here")


# speedup vs baseline: 193.8604x; 193.8604x over previous
import functools

import jax
import jax.numpy as jnp
from jax import lax
from jax.experimental import pallas as pl
from jax.experimental.pallas import tpu as pltpu
import jax.experimental.pallas.tpu_sc as plsc

N = 50000
E = 800000
T = 4
IN = 8
OUT = 8
B = 4
D = B * OUT
NC = 2
NS = 16
NW = NC * NS
KB = 128
NBLK = 196
CBLK = 28
NCHUNK = NBLK // CBLK
EPT = NBLK * KB
EP = NW * EPT
NROWSP = 51200
RPT = NROWSP // NS
ZR = 160
NB_TC = 1000
GRID = N // NB_TC
K1 = 40


def _xw_body(x_ref, w_ref, o_ref):
    o_ref[...] = jnp.dot(x_ref[...], w_ref[...],
                         preferred_element_type=jnp.float32)


def _xw_call(xt40, wmat40):
    return pl.pallas_call(
        _xw_body,
        grid=(GRID,),
        in_specs=[pl.BlockSpec((NB_TC, K1), lambda i: (i, 0)),
                  pl.BlockSpec((K1, T * D), lambda i: (0, 0))],
        out_specs=pl.BlockSpec((NB_TC, T * D), lambda i: (i, 0)),
        out_shape=jax.ShapeDtypeStruct((N, T * D), jnp.float32),
    )(xt40, wmat40)


_mesh = plsc.VectorSubcoreMesh(core_axis_name="c", subcore_axis_name="s")


@functools.partial(
    pl.kernel,
    out_type=jax.ShapeDtypeStruct((NC, NROWSP, D), jnp.float32),
    mesh=_mesh,
    scratch_types=[
        pltpu.VMEM((CBLK, KB), jnp.int32),
        pltpu.VMEM((CBLK, KB), jnp.int32),
        pltpu.VMEM((KB, D), jnp.float32),
        pltpu.VMEM((KB, D), jnp.float32),
        pltpu.VMEM((ZR, D), jnp.float32),
        pltpu.VMEM_SHARED((NROWSP, D), jnp.float32),
        pltpu.SemaphoreType.DMA,
        pltpu.SemaphoreType.DMA,
    ],
    compiler_params=pltpu.CompilerParams(use_tc_tiling_on_sc=False),
)
def _sc_accum(xw_hbm, g_hbm, d_hbm, y_hbm,
              g_v, d_v, ra_v, rb_v, zb_v, acc_sh, sem_a, sem_b):
    c = lax.axis_index("c")
    s = lax.axis_index("s")
    wid = c * NS + s
    z16 = jnp.zeros((16,), jnp.float32)

    def _zrow(i, carry):
        zb_v[i, pl.ds(0, 16)] = z16
        zb_v[i, pl.ds(16, 16)] = z16
        return carry

    lax.fori_loop(0, ZR, _zrow, 0)

    base = s * RPT
    for j in range(RPT // ZR):
        pltpu.sync_copy(zb_v, acc_sh.at[pl.ds(base + j * ZR, ZR)])
    plsc.subcore_barrier()

    def _chunk(q, carry):
        pltpu.sync_copy(g_hbm.at[wid, q], g_v)
        pltpu.sync_copy(d_hbm.at[wid, q], d_v)
        pltpu.async_copy(xw_hbm.at[g_v.at[0]], ra_v, sem_a)

        def _pair(p, inner):
            j0 = 2 * p
            pltpu.make_async_copy(xw_hbm.at[g_v.at[j0]], ra_v, sem_a).wait()
            pltpu.async_copy(xw_hbm.at[g_v.at[j0 + 1]], rb_v, sem_b)
            pltpu.sync_copy(ra_v, acc_sh.at[d_v.at[j0]], add=True)
            pltpu.make_async_copy(xw_hbm.at[g_v.at[j0 + 1]], rb_v, sem_b).wait()

            @pl.when(j0 + 2 < CBLK)
            def _():
                pltpu.async_copy(xw_hbm.at[g_v.at[j0 + 2]], ra_v, sem_a)

            pltpu.sync_copy(rb_v, acc_sh.at[d_v.at[j0 + 1]], add=True)
            return inner

        lax.fori_loop(0, CBLK // 2, _pair, 0)
        return carry

    lax.fori_loop(0, NCHUNK, _chunk, 0)
    plsc.subcore_barrier()
    pltpu.sync_copy(acc_sh.at[pl.ds(base, RPT)],
                    y_hbm.at[c, pl.ds(base, RPT)])


def _fin_body(y_ref, o_ref):
    ysum = y_ref[0] + y_ref[1]
    for b in range(B):
        o_ref[b] = ysum[:, b * OUT:(b + 1) * OUT]


def _fin_call(ypart):
    return pl.pallas_call(
        _fin_body,
        grid=(GRID,),
        in_specs=[pl.BlockSpec((NC, NB_TC, D), lambda i: (0, i, 0))],
        out_specs=pl.BlockSpec((B, NB_TC, OUT), lambda i: (0, i, 0)),
        out_shape=jax.ShapeDtypeStruct((B, N, OUT), jnp.float32),
    )(ypart)


def kernel(input, edge_index, edge_types, weight_vector, bias_vector):
    x = input.astype(jnp.float32)
    W = weight_vector.reshape(T, OUT, IN)
    bias = bias_vector.reshape(T, OUT)

    Wt = jnp.transpose(W, (2, 0, 1))
    eye = jnp.eye(B, dtype=jnp.float32)
    wfull = eye[:, None, None, :, None] * Wt[None, :, :, None, :]
    wmat = wfull.reshape(B * IN, T * B * OUT)
    brow = jnp.broadcast_to(bias[:, None, :], (T, B, OUT)).reshape(1, T * B * OUT)
    wmat40 = jnp.concatenate(
        [wmat, brow,
         jnp.zeros((K1 - B * IN - 1, T * B * OUT), jnp.float32)], axis=0)

    xt = jnp.transpose(x, (1, 0, 2)).reshape(N, B * IN)
    xt40 = jnp.concatenate(
        [xt, jnp.ones((N, 1), jnp.float32),
         jnp.zeros((N, K1 - B * IN - 1), jnp.float32)], axis=1)

    xw = _xw_call(xt40, wmat40).reshape(N * T, D)

    src = edge_index[1].astype(jnp.int32)
    dst = edge_index[0].astype(jnp.int32)
    typ = edge_types.astype(jnp.int32)
    g = src * T + typ
    gp = jnp.concatenate(
        [g, jnp.zeros((EP - E,), jnp.int32)]).reshape(NW, NCHUNK, CBLK, KB)
    dp = jnp.concatenate(
        [dst, jnp.full((EP - E,), N, jnp.int32)]).reshape(NW, NCHUNK, CBLK, KB)

    ypart = _sc_accum(xw, gp, dp)
    return _fin_call(ypart)


# speedup vs baseline: 229.7097x; 1.1849x over previous
import functools

import jax
import jax.numpy as jnp
from jax import lax
from jax.experimental import pallas as pl
from jax.experimental.pallas import tpu as pltpu
import jax.experimental.pallas.tpu_sc as plsc

N = 50000
E = 800000
T = 4
IN = 8
OUT = 8
B = 4
D = B * OUT
NC = 2
NS = 16
NW = NC * NS
KB = 128
NBLK = 196
CBLK = 28
NCHUNK = NBLK // CBLK
EPT = NBLK * KB
EP = NW * EPT
NROWSP = 50048
RPT = NROWSP // NS
ZR = 136
NBUF = 4
NROWS128 = NROWSP * D // 128
RB = NROWS128 // 4
NB_TC = 1000
GRID = N // NB_TC
K1 = 40


def _xw_body(x_ref, w_ref, o_ref):
    o_ref[...] = jnp.dot(x_ref[...], w_ref[...],
                         preferred_element_type=jnp.float32)


def _xw_call(xt40, wmat40):
    return pl.pallas_call(
        _xw_body,
        grid=(GRID,),
        in_specs=[pl.BlockSpec((NB_TC, K1), lambda i: (i, 0)),
                  pl.BlockSpec((K1, T * D), lambda i: (0, 0))],
        out_specs=pl.BlockSpec((NB_TC, T * D), lambda i: (i, 0)),
        out_shape=jax.ShapeDtypeStruct((N, T * D), jnp.float32),
    )(xt40, wmat40)


_mesh = plsc.VectorSubcoreMesh(core_axis_name="c", subcore_axis_name="s")


@functools.partial(
    pl.kernel,
    out_type=jax.ShapeDtypeStruct((NC, NROWSP, D), jnp.float32),
    mesh=_mesh,
    scratch_types=[
        pltpu.VMEM((CBLK, KB), jnp.int32),
        pltpu.VMEM((CBLK, KB), jnp.int32),
        [pltpu.VMEM((KB, D), jnp.float32) for _ in range(NBUF)],
        pltpu.VMEM((ZR, D), jnp.float32),
        pltpu.VMEM_SHARED((NROWSP, D), jnp.float32),
        [pltpu.SemaphoreType.DMA for _ in range(NBUF)],
    ],
    compiler_params=pltpu.CompilerParams(use_tc_tiling_on_sc=False),
)
def _sc_accum(xw_hbm, g_hbm, d_hbm, y_hbm,
              g_v, d_v, rows_v, zb_v, acc_sh, sems):
    c = lax.axis_index("c")
    s = lax.axis_index("s")
    wid = c * NS + s
    z16 = jnp.zeros((16,), jnp.float32)

    def _zrow(i, carry):
        zb_v[i, pl.ds(0, 16)] = z16
        zb_v[i, pl.ds(16, 16)] = z16
        return carry

    lax.fori_loop(0, ZR, _zrow, 0)

    base = s * RPT
    for j in range(RPT // ZR):
        pltpu.sync_copy(zb_v, acc_sh.at[pl.ds(base + j * ZR, ZR)])
    plsc.subcore_barrier()

    def _chunk(q, carry):
        pltpu.sync_copy(g_hbm.at[wid, q], g_v)
        pltpu.sync_copy(d_hbm.at[wid, q], d_v)
        for i in range(NBUF):
            pltpu.async_copy(xw_hbm.at[g_v.at[i]], rows_v[i], sems[i])

        def _group(p, inner):
            for i in range(NBUF):
                j = NBUF * p + i
                pltpu.make_async_copy(
                    xw_hbm.at[g_v.at[j]], rows_v[i], sems[i]).wait()
                pltpu.sync_copy(rows_v[i], acc_sh.at[d_v.at[j]], add=True)

                @pl.when(j + NBUF < CBLK)
                def _():
                    pltpu.async_copy(
                        xw_hbm.at[g_v.at[j + NBUF]], rows_v[i], sems[i])
            return inner

        lax.fori_loop(0, CBLK // NBUF, _group, 0)
        return carry

    lax.fori_loop(0, NCHUNK, _chunk, 0)
    plsc.subcore_barrier()
    pltpu.sync_copy(acc_sh.at[pl.ds(base, RPT)],
                    y_hbm.at[c, pl.ds(base, RPT)])


def _fin_body(y_ref, o_ref):
    o_ref[...] = y_ref[0] + y_ref[1]


def _fin_call(ypart128):
    return pl.pallas_call(
        _fin_body,
        grid=(NROWS128 // RB,),
        in_specs=[pl.BlockSpec((NC, RB, 128), lambda i: (0, i, 0))],
        out_specs=pl.BlockSpec((RB, 128), lambda i: (i, 0)),
        out_shape=jax.ShapeDtypeStruct((NROWS128, 128), jnp.float32),
    )(ypart128)


def kernel(input, edge_index, edge_types, weight_vector, bias_vector):
    x = input.astype(jnp.float32)
    W = weight_vector.reshape(T, OUT, IN)
    bias = bias_vector.reshape(T, OUT)

    Wt = jnp.transpose(W, (2, 0, 1))
    eye = jnp.eye(B, dtype=jnp.float32)
    wfull = eye[:, None, None, :, None] * Wt[None, :, :, None, :]
    wmat = wfull.reshape(B * IN, T * B * OUT)
    brow = jnp.broadcast_to(bias[:, None, :], (T, B, OUT)).reshape(1, T * B * OUT)
    wmat40 = jnp.concatenate(
        [wmat, brow,
         jnp.zeros((K1 - B * IN - 1, T * B * OUT), jnp.float32)], axis=0)

    xt = jnp.transpose(x, (1, 0, 2)).reshape(N, B * IN)
    xt40 = jnp.concatenate(
        [xt, jnp.ones((N, 1), jnp.float32),
         jnp.zeros((N, K1 - B * IN - 1), jnp.float32)], axis=1)

    xw = _xw_call(xt40, wmat40).reshape(N * T, D)

    src = edge_index[1].astype(jnp.int32)
    dst = edge_index[0].astype(jnp.int32)
    typ = edge_types.astype(jnp.int32)
    g = src * T + typ
    gp = jnp.concatenate(
        [g, jnp.zeros((EP - E,), jnp.int32)]).reshape(NW, NCHUNK, CBLK, KB)
    dp = jnp.concatenate(
        [dst, jnp.full((EP - E,), N, jnp.int32)]).reshape(NW, NCHUNK, CBLK, KB)

    ypart = _sc_accum(xw, gp, dp)
    ysum = _fin_call(ypart.reshape(NC, NROWS128, 128))
    y = ysum.reshape(NROWSP, B, OUT)[:N]
    return jnp.transpose(y, (1, 0, 2))
